# Initial kernel scaffold; baseline (speedup 1.0000x reference)
import jax
import jax.numpy as jnp
from jax.experimental import pallas as pl


def kernel(x, m, table, W, b):
    raise NotImplementedError("write your pallas kernel here")


# speedup vs baseline: 6.1861x; 6.1861x over previous
import functools

import jax
import jax.numpy as jnp
from jax import lax
from jax.experimental import pallas as pl
from jax.experimental.pallas import tpu as pltpu
from jax.experimental.pallas import tpu_sc as plsc

VOCAB = 100000
DIM = 128
LABELS = 1000
B = 4096
L = 200

NC = 2
NS = 16
NW = NC * NS
DOCS_PER_W = B // NW
LANES = 16
NSEG = DIM // LANES


def _sc_gather_sum(x, table):
    mesh = plsc.VectorSubcoreMesh(core_axis_name="c", subcore_axis_name="s")

    @functools.partial(
        pl.kernel,
        mesh=mesh,
        out_type=jax.ShapeDtypeStruct((B, DIM), jnp.float32),
        scratch_types=[
            pltpu.VMEM((L,), jnp.int32),
            pltpu.VMEM((L, DIM), jnp.float32),
            pltpu.VMEM((DOCS_PER_W, DIM), jnp.float32),
            pltpu.SemaphoreType.DMA,
        ],
    )
    def k(x_hbm, table_hbm, out_hbm, idx_v, rows_v, out_v, sem):
        wid = lax.axis_index("s") * NC + lax.axis_index("c")
        base = wid * DOCS_PER_W

        @pl.loop(0, DOCS_PER_W)
        def _(i):
            d = base + i
            pltpu.sync_copy(x_hbm.at[d], idx_v)
            c1 = pltpu.async_copy(
                table_hbm.at[idx_v.at[pl.ds(0, 128)]],
                rows_v.at[pl.ds(0, 128)], sem)
            c2 = pltpu.async_copy(
                table_hbm.at[idx_v.at[pl.ds(128, L - 128)]],
                rows_v.at[pl.ds(128, L - 128)], sem)
            c1.wait()
            c2.wait()

            zero = jnp.zeros((LANES,), jnp.float32)

            @plsc.parallel_loop(0, L, 1, unroll=4, carry=(zero,) * NSEG)
            def acc(r, carry):
                return tuple(
                    carry[j] + rows_v[r, pl.ds(j * LANES, LANES)]
                    for j in range(NSEG))
            for j in range(NSEG):
                out_v[i, pl.ds(j * LANES, LANES)] = acc[j]

        pltpu.sync_copy(out_v, out_hbm.at[pl.ds(base, DOCS_PER_W)])

    return k(x, table)


def _tc_binarize_matmul(doc_sum, W, b):
    LB = 1024
    Wp = jnp.zeros((LB, DIM), jnp.float32).at[:LABELS].set(W)
    bp = jnp.zeros((1, LB), jnp.float32).at[0, :LABELS].set(b)
    BBLK = 512

    def body(e_ref, w_ref, b_ref, o_ref):
        e = (e_ref[...] > 0.0).astype(jnp.float32)
        o_ref[...] = lax.dot_general(
            e, w_ref[...], (((1,), (1,)), ((), ())),
            preferred_element_type=jnp.float32,
            precision=lax.Precision.HIGHEST) + b_ref[...]

    out = pl.pallas_call(
        body,
        grid=(B // BBLK,),
        in_specs=[
            pl.BlockSpec((BBLK, DIM), lambda i: (i, 0)),
            pl.BlockSpec((LB, DIM), lambda i: (0, 0)),
            pl.BlockSpec((1, LB), lambda i: (0, 0)),
        ],
        out_specs=pl.BlockSpec((BBLK, LB), lambda i: (i, 0)),
        out_shape=jax.ShapeDtypeStruct((B, LB), jnp.float32),
    )(doc_sum, Wp, bp)
    return out[:, :LABELS]


def kernel(x, m, table, W, b):
    del m
    doc_sum = _sc_gather_sum(x, table)
    return _tc_binarize_matmul(doc_sum, W, b)


# speedup vs baseline: 12.0011x; 1.9400x over previous
import functools

import jax
import jax.numpy as jnp
from jax import lax
from jax.experimental import pallas as pl
from jax.experimental.pallas import tpu as pltpu
from jax.experimental.pallas import tpu_sc as plsc

VOCAB = 100000
DIM = 128
LABELS = 1000
B = 4096
L = 200

NC = 2
NS = 16
NW = NC * NS
DOCS_PER_W = B // NW
LANES = 16
NSEG = DIM // LANES


def _sc_gather_sum(x, table):
    mesh = plsc.VectorSubcoreMesh(core_axis_name="c", subcore_axis_name="s")

    @functools.partial(
        pl.kernel,
        mesh=mesh,
        out_type=jax.ShapeDtypeStruct((B, DIM), jnp.float32),
        scratch_types=[
            pltpu.VMEM((DOCS_PER_W, L), jnp.int32),
            pltpu.VMEM((L, DIM), jnp.float32),
            pltpu.VMEM((L, DIM), jnp.float32),
            pltpu.VMEM((DOCS_PER_W, DIM), jnp.float32),
            pltpu.SemaphoreType.DMA,
            pltpu.SemaphoreType.DMA,
        ],
    )
    def k(x_hbm, table_hbm, out_hbm, idx_v, rows0, rows1, out_v, s0, s1):
        wid = lax.axis_index("s") * NC + lax.axis_index("c")
        base = wid * DOCS_PER_W

        pltpu.sync_copy(x_hbm.at[pl.ds(base, DOCS_PER_W)], idx_v)

        def fire(d, buf, sem):
            pltpu.async_copy(table_hbm.at[idx_v.at[d, pl.ds(0, 128)]],
                             buf.at[pl.ds(0, 128)], sem)
            pltpu.async_copy(table_hbm.at[idx_v.at[d, pl.ds(128, L - 128)]],
                             buf.at[pl.ds(128, L - 128)], sem)

        def drain(buf, sem):
            pltpu.make_async_copy(table_hbm.at[pl.ds(0, 128)],
                                  buf.at[pl.ds(0, 128)], sem).wait()
            pltpu.make_async_copy(table_hbm.at[pl.ds(0, L - 128)],
                                  buf.at[pl.ds(128, L - 128)], sem).wait()

        def reduce(d, buf):
            zero = jnp.zeros((LANES,), jnp.float32)

            @plsc.parallel_loop(0, L, 1, unroll=4, carry=(zero,) * NSEG)
            def acc(r, carry):
                return tuple(
                    carry[j] + buf[r, pl.ds(j * LANES, LANES)]
                    for j in range(NSEG))
            for j in range(NSEG):
                out_v[d, pl.ds(j * LANES, LANES)] = acc[j]

        fire(0, rows0, s0)

        @pl.loop(0, DOCS_PER_W // 2)
        def _(g):
            a = 2 * g
            fire(a + 1, rows1, s1)
            drain(rows0, s0)
            reduce(a, rows0)

            @pl.when(g < DOCS_PER_W // 2 - 1)
            def _():
                fire(a + 2, rows0, s0)
            drain(rows1, s1)
            reduce(a + 1, rows1)

        pltpu.sync_copy(out_v, out_hbm.at[pl.ds(base, DOCS_PER_W)])

    return k(x, table)


def _tc_binarize_matmul(doc_sum, W, b):
    LB = 1024
    Wp = jnp.zeros((LB, DIM), jnp.float32).at[:LABELS].set(W)
    bp = jnp.zeros((1, LB), jnp.float32).at[0, :LABELS].set(b)
    BBLK = 512

    def body(e_ref, w_ref, b_ref, o_ref):
        e = (e_ref[...] > 0.0).astype(jnp.float32)
        o_ref[...] = lax.dot_general(
            e, w_ref[...], (((1,), (1,)), ((), ())),
            preferred_element_type=jnp.float32,
            precision=lax.Precision.HIGHEST) + b_ref[...]

    out = pl.pallas_call(
        body,
        grid=(B // BBLK,),
        in_specs=[
            pl.BlockSpec((BBLK, DIM), lambda i: (i, 0)),
            pl.BlockSpec((LB, DIM), lambda i: (0, 0)),
            pl.BlockSpec((1, LB), lambda i: (0, 0)),
        ],
        out_specs=pl.BlockSpec((BBLK, LB), lambda i: (i, 0)),
        out_shape=jax.ShapeDtypeStruct((B, LB), jnp.float32),
    )(doc_sum, Wp, bp)
    return out[:, :LABELS]


def kernel(x, m, table, W, b):
    del m
    doc_sum = _sc_gather_sum(x, table)
    return _tc_binarize_matmul(doc_sum, W, b)
